# Initial kernel scaffold; baseline (speedup 1.0000x reference)
import jax
import jax.numpy as jnp
from jax.experimental import pallas as pl


def kernel(evidence_type, W):
    raise NotImplementedError("write your pallas kernel here")


# speedup vs baseline: 12.4527x; 12.4527x over previous
import functools

import jax
import jax.numpy as jnp
from jax import lax
from jax.experimental import pallas as pl
from jax.experimental.pallas import tpu as pltpu

B = 1
S = 2048
REL_NUM = 16
HEAD = 64
ROWS_PER_BLOCK = 8


def _fanout_body(e_sm, e_col_ref, w_ref, out_ref, slab_ref):
    i = pl.program_id(0)

    @pl.when(i == 0)
    def _build_slabs():
        e_col = e_col_ref[...]
        iota_r = lax.broadcasted_iota(jnp.int32, (S, REL_NUM), 1)
        w = w_ref[...]
        for a in range(4):
            onehot = (iota_r == (e_col + 4 * a)).astype(jnp.float32)
            slab_ref[a] = jnp.dot(
                onehot, w, preferred_element_type=jnp.float32
            )

    for k in range(ROWS_PER_BLOCK):
        a_k = e_sm[i * ROWS_PER_BLOCK + k]
        out_ref[k] = slab_ref[a_k]


@jax.jit
def kernel(evidence_type, W):
    e = evidence_type.reshape(S).astype(jnp.int32)
    e_col = e.reshape(S, 1)

    grid_spec = pltpu.PrefetchScalarGridSpec(
        num_scalar_prefetch=1,
        grid=(S // ROWS_PER_BLOCK,),
        in_specs=[
            pl.BlockSpec((S, 1), lambda i, e_sm: (0, 0)),
            pl.BlockSpec((REL_NUM, HEAD), lambda i, e_sm: (0, 0)),
        ],
        out_specs=pl.BlockSpec(
            (ROWS_PER_BLOCK, S, HEAD), lambda i, e_sm: (i, 0, 0)
        ),
        scratch_shapes=[pltpu.VMEM((4, S, HEAD), jnp.float32)],
    )

    out = pl.pallas_call(
        _fanout_body,
        grid_spec=grid_spec,
        out_shape=jax.ShapeDtypeStruct((S, S, HEAD), jnp.float32),
    )(e, e_col, W)
    return out.reshape(B, S, S, HEAD)
